# Initial kernel scaffold; baseline (speedup 1.0000x reference)
import jax
import jax.numpy as jnp
from jax.experimental import pallas as pl


def kernel(polygon_center, polygon_type, polygon_on_route, polygon_tl_status, polygon_has_speed_limit, polygon_speed_limit, point_position, point_vector, point_orientation, polygon_orientation, valid_mask, pe_W1, pe_b1, pe_W2, pe_b2, pe_W3, pe_b3, pe_W4, pe_b4, sl_W1, sl_b1, sl_W2, sl_b2, type_emb, on_route_emb, tl_emb, unk_emb):
    raise NotImplementedError("write your pallas kernel here")


# speedup vs baseline: 1.4583x; 1.4583x over previous
import functools

import jax
import jax.numpy as jnp
from jax.experimental import pallas as pl

BS, M, P, DIM = 32, 128, 20, 128
N = BS * M
BLK = 256


def _fused_kernel(x_ref, mask_ref, scal_ref, w1_ref, b1_ref, w2_ref, b2_ref,
                  w3t_ref, w3b_ref, b3_ref, w4_ref, b4_ref,
                  slw1_ref, slb1_ref, slw2_ref, slb2_ref, emb_ref, o_ref):
    f32 = jnp.float32
    w1 = w1_ref[...]
    b1 = b1_ref[...]
    w2 = w2_ref[...]
    b2 = b2_ref[...]

    h2s = []
    pooled = None
    for p in range(P):
        x = x_ref[p]
        h1 = jax.nn.relu(jnp.dot(x, w1, preferred_element_type=f32) + b1)
        h2 = jnp.dot(h1, w2, preferred_element_type=f32) + b2
        h2 = h2 * mask_ref[:, p:p + 1]
        h2s.append(h2)
        pooled = h2 if pooled is None else jnp.maximum(pooled, h2)

    pb = jnp.dot(pooled, w3b_ref[...], preferred_element_type=f32) + b3_ref[...]

    w3t = w3t_ref[...]
    w4 = w4_ref[...]
    b4 = b4_ref[...]
    out = None
    for p in range(P):
        g1 = jax.nn.relu(jnp.dot(h2s[p], w3t, preferred_element_type=f32) + pb)
        g = jnp.dot(g1, w4, preferred_element_type=f32) + b4
        g = g * mask_ref[:, p:p + 1]
        out = g if out is None else jnp.maximum(out, g)

    t = scal_ref[:, 0:1]
    r = scal_ref[:, 1:2]
    tl = scal_ref[:, 2:3]
    has = scal_ref[:, 3:4]
    s = scal_ref[:, 4:5]
    iota = jax.lax.broadcasted_iota(jnp.int32, (BLK, 16), 1).astype(f32)
    onehot = ((iota == t).astype(f32) + (iota == r + 3.0).astype(f32)
              + (iota == tl + 5.0).astype(f32)
              + (iota == 9.0).astype(f32) * (1.0 - has))
    cat = jnp.dot(onehot, emb_ref[...], preferred_element_type=f32)

    hs = jax.nn.relu(s * slw1_ref[...] + slb1_ref[...])
    sl = jnp.dot(hs, slw2_ref[...], preferred_element_type=f32) + slb2_ref[...]
    o_ref[...] = out + cat + sl * has


@functools.partial(jax.jit, static_argnames=())
def kernel(polygon_center, polygon_type, polygon_on_route, polygon_tl_status,
           polygon_has_speed_limit, polygon_speed_limit, point_position,
           point_vector, point_orientation, polygon_orientation, valid_mask,
           pe_W1, pe_b1, pe_W2, pe_b2, pe_W3, pe_b3, pe_W4, pe_b4,
           sl_W1, sl_b1, sl_W2, sl_b2, type_emb, on_route_emb, tl_emb, unk_emb):
    f32 = jnp.float32
    bn = jnp.asarray(1.0 / jnp.sqrt(1.0 + 1e-5), f32)

    pp = point_position[:, :, 0].reshape(N, P, 2)
    pv = point_vector[:, :, 0].reshape(N, P, 2)
    po = point_orientation[:, :, 0].reshape(N, P)
    c2 = polygon_center[..., :2].reshape(N, 1, 2)
    feat = jnp.concatenate(
        [pp - c2, pv, jnp.cos(po)[..., None], jnp.sin(po)[..., None],
         jnp.zeros((N, P, 2), f32)], axis=-1)
    x = jnp.transpose(feat, (1, 0, 2))
    mask = valid_mask.reshape(N, P).astype(f32)
    scal = jnp.stack(
        [polygon_type.reshape(N).astype(f32),
         polygon_on_route.reshape(N).astype(f32),
         polygon_tl_status.reshape(N).astype(f32),
         polygon_has_speed_limit.reshape(N).astype(f32),
         polygon_speed_limit.reshape(N),
         jnp.zeros((N,), f32), jnp.zeros((N,), f32), jnp.zeros((N,), f32)],
        axis=-1)

    w1 = jnp.concatenate([pe_W1, jnp.zeros((2, 128), f32)], axis=0) * bn
    b1 = (pe_b1 * bn).reshape(1, 128)
    w3t = pe_W3[:256] * bn
    w3b = pe_W3[256:] * bn
    b3 = (pe_b3 * bn).reshape(1, 256)
    emb = jnp.concatenate(
        [type_emb, on_route_emb, tl_emb, unk_emb,
         jnp.zeros((6, DIM), f32)], axis=0)

    grid = (N // BLK,)
    rep = lambda shape: pl.BlockSpec(shape, lambda i: (0,) * len(shape))
    out = pl.pallas_call(
        _fused_kernel,
        grid=grid,
        in_specs=[
            pl.BlockSpec((P, BLK, 8), lambda i: (0, i, 0)),
            pl.BlockSpec((BLK, P), lambda i: (i, 0)),
            pl.BlockSpec((BLK, 8), lambda i: (i, 0)),
            rep((8, 128)), rep((1, 128)),
            rep((128, 256)), rep((1, 256)),
            rep((256, 256)), rep((256, 256)), rep((1, 256)),
            rep((256, 128)), rep((1, 128)),
            rep((1, 128)), rep((1, 128)), rep((128, 128)), rep((1, 128)),
            rep((16, 128)),
        ],
        out_specs=pl.BlockSpec((BLK, DIM), lambda i: (i, 0)),
        out_shape=jax.ShapeDtypeStruct((N, DIM), f32),
    )(x, mask, scal, w1, b1, pe_W2, pe_b2.reshape(1, 256), w3t, w3b, b3,
      pe_W4, pe_b4.reshape(1, 128), sl_W1, sl_b1.reshape(1, 128), sl_W2,
      sl_b2.reshape(1, 128), emb)
    return out.reshape(BS, M, DIM)


# speedup vs baseline: 1.5222x; 1.0438x over previous
import functools

import jax
import jax.numpy as jnp
from jax.experimental import pallas as pl

BS, M, P, DIM = 32, 128, 20, 128
N = BS * M
BLK = 256


def _fused_kernel(x_ref, mask_ref, scal_ref, w1_ref, b1_ref, w2_ref, b2_ref,
                  w3t_ref, w3b_ref, b3_ref, w4_ref, b4_ref,
                  slw1_ref, slb1_ref, slw2_ref, slb2_ref, emb_ref, o_ref):
    f32 = jnp.float32
    w1 = w1_ref[...]
    b1 = b1_ref[...]
    w2 = w2_ref[...]
    b2 = b2_ref[...]

    h2s = []
    pooled = None
    for p in range(P):
        x = x_ref[:, 8 * p:8 * p + 8]
        h1 = jax.nn.relu(jnp.dot(x, w1, preferred_element_type=f32) + b1)
        h2 = jnp.dot(h1, w2, preferred_element_type=f32) + b2
        h2 = h2 * mask_ref[:, p:p + 1]
        h2s.append(h2)
        pooled = h2 if pooled is None else jnp.maximum(pooled, h2)

    pb = jnp.dot(pooled, w3b_ref[...], preferred_element_type=f32) + b3_ref[...]

    w3t = w3t_ref[...]
    w4 = w4_ref[...]
    b4 = b4_ref[...]
    out = None
    for p in range(P):
        g1 = jax.nn.relu(jnp.dot(h2s[p], w3t, preferred_element_type=f32) + pb)
        g = jnp.dot(g1, w4, preferred_element_type=f32) + b4
        g = g * mask_ref[:, p:p + 1]
        out = g if out is None else jnp.maximum(out, g)

    t = scal_ref[:, 0:1]
    r = scal_ref[:, 1:2]
    tl = scal_ref[:, 2:3]
    has = scal_ref[:, 3:4]
    s = scal_ref[:, 4:5]
    iota = jax.lax.broadcasted_iota(jnp.int32, (BLK, 16), 1).astype(f32)
    onehot = ((iota == t).astype(f32) + (iota == r + 3.0).astype(f32)
              + (iota == tl + 5.0).astype(f32)
              + (iota == 9.0).astype(f32) * (1.0 - has))
    cat = jnp.dot(onehot, emb_ref[...], preferred_element_type=f32)

    hs = jax.nn.relu(s * slw1_ref[...] + slb1_ref[...])
    sl = jnp.dot(hs, slw2_ref[...], preferred_element_type=f32) + slb2_ref[...]
    o_ref[...] = out + cat + sl * has


@functools.partial(jax.jit, static_argnames=())
def kernel(polygon_center, polygon_type, polygon_on_route, polygon_tl_status,
           polygon_has_speed_limit, polygon_speed_limit, point_position,
           point_vector, point_orientation, polygon_orientation, valid_mask,
           pe_W1, pe_b1, pe_W2, pe_b2, pe_W3, pe_b3, pe_W4, pe_b4,
           sl_W1, sl_b1, sl_W2, sl_b2, type_emb, on_route_emb, tl_emb, unk_emb):
    f32 = jnp.float32
    bn = jnp.asarray(1.0 / jnp.sqrt(1.0 + 1e-5), f32)

    pp = point_position[:, :, 0].reshape(N, P, 2)
    pv = point_vector[:, :, 0].reshape(N, P, 2)
    po = point_orientation[:, :, 0].reshape(N, P)
    c2 = polygon_center[..., :2].reshape(N, 1, 2)
    feat = jnp.concatenate(
        [pp - c2, pv, jnp.cos(po)[..., None], jnp.sin(po)[..., None],
         jnp.zeros((N, P, 2), f32)], axis=-1)
    x = feat.reshape(N, P * 8)
    mask = valid_mask.reshape(N, P).astype(f32)
    scal = jnp.stack(
        [polygon_type.reshape(N).astype(f32),
         polygon_on_route.reshape(N).astype(f32),
         polygon_tl_status.reshape(N).astype(f32),
         polygon_has_speed_limit.reshape(N).astype(f32),
         polygon_speed_limit.reshape(N),
         jnp.zeros((N,), f32), jnp.zeros((N,), f32), jnp.zeros((N,), f32)],
        axis=-1)

    w1 = jnp.concatenate([pe_W1, jnp.zeros((2, 128), f32)], axis=0) * bn
    b1 = (pe_b1 * bn).reshape(1, 128)
    w3t = pe_W3[:256] * bn
    w3b = pe_W3[256:] * bn
    b3 = (pe_b3 * bn).reshape(1, 256)
    emb = jnp.concatenate(
        [type_emb, on_route_emb, tl_emb, unk_emb,
         jnp.zeros((6, DIM), f32)], axis=0)

    grid = (N // BLK,)
    rep = lambda shape: pl.BlockSpec(shape, lambda i: (0,) * len(shape))
    out = pl.pallas_call(
        _fused_kernel,
        grid=grid,
        in_specs=[
            pl.BlockSpec((BLK, P * 8), lambda i: (i, 0)),
            pl.BlockSpec((BLK, P), lambda i: (i, 0)),
            pl.BlockSpec((BLK, 8), lambda i: (i, 0)),
            rep((8, 128)), rep((1, 128)),
            rep((128, 256)), rep((1, 256)),
            rep((256, 256)), rep((256, 256)), rep((1, 256)),
            rep((256, 128)), rep((1, 128)),
            rep((1, 128)), rep((1, 128)), rep((128, 128)), rep((1, 128)),
            rep((16, 128)),
        ],
        out_specs=pl.BlockSpec((BLK, DIM), lambda i: (i, 0)),
        out_shape=jax.ShapeDtypeStruct((N, DIM), f32),
    )(x, mask, scal, w1, b1, pe_W2, pe_b2.reshape(1, 256), w3t, w3b, b3,
      pe_W4, pe_b4.reshape(1, 128), sl_W1, sl_b1.reshape(1, 128), sl_W2,
      sl_b2.reshape(1, 128), emb)
    return out.reshape(BS, M, DIM)
